# Initial kernel scaffold; baseline (speedup 1.0000x reference)
import functools

import jax
import jax.numpy as jnp
from jax import lax
from jax.experimental import pallas as pl
from jax.experimental.pallas import tpu as pltpu
from jax.experimental.pallas import tpu_sc as plsc

G = 256
ND = 128
ED = 16
PD = 128
N = 10000
N_PAD = 10240
E = 320000

NC, NS = 2, 16
NW = NC * NS

EC = 1280
E_CHUNKS = E // EC
E_K = (E_CHUNKS + NW - 1) // NW
NCH = 160
N_CHUNKS = N_PAD // NCH


def _sc_body(ef, eids, nf, nids, wge, bge, wgn, bgn,
             npart, epart,
             ebuf, egbuf, eidb, nbuf, ngbuf, nidb,
             wgeb, bgeb, wgnb, bgnb,
             npool, epool):
  c = lax.axis_index("c")
  s = lax.axis_index("s")
  wid = s * NC + c

  pltpu.sync_copy(wge, wgeb)
  pltpu.sync_copy(bge, bgeb)
  pltpu.sync_copy(wgn, wgnb)
  pltpu.sync_copy(bgn, bgnb)

  @pl.when(s == 0)
  def _zero():
    zero16 = jnp.zeros((16,), jnp.float32)

    def zrow_n(i, carry):
      for cc in range(8):
        ngbuf[i, cc * 16:(cc + 1) * 16] = zero16
      return carry

    lax.fori_loop(0, NCH, zrow_n, 0)

    def zrow_e(i, carry):
      egbuf[i, :] = zero16
      return carry

    lax.fori_loop(0, G, zrow_e, 0)

    pltpu.sync_copy(ngbuf, npool.at[pl.ds(0, NCH)])
    pltpu.sync_copy(ngbuf.at[pl.ds(0, G - NCH)], npool.at[pl.ds(NCH, G - NCH)])
    pltpu.sync_copy(egbuf.at[pl.ds(0, G)], epool)

  plsc.subcore_barrier()

  wgev = wgeb[:]
  bgev = bgeb[:]

  for k in range(E_K):
    cid = wid + NW * k

    @pl.when(cid < E_CHUNKS)
    def _edge_chunk():
      base = cid * EC
      pltpu.sync_copy(ef.at[pl.ds(base, EC)], ebuf)
      pltpu.sync_copy(eids.at[pl.ds(cid * 10, 10)], eidb)

      def gate(j, carry):
        row = ebuf[j, :]
        a = jnp.sum(row * wgev)
        egbuf[j, :] = row * a + row * bgev
        return carry

      lax.fori_loop(0, EC, gate, 0)
      for r in range(10):
        pltpu.sync_copy(egbuf.at[pl.ds(r * 128, 128)],
                        epool.at[eidb.at[r]], add=True)

  wgnv = [wgnb[cc * 16:(cc + 1) * 16] for cc in range(8)]
  bgnv = bgnb[:]
  for k in range(2):
    ncid = wid * 2 + k
    base = ncid * NCH
    pltpu.sync_copy(nf.at[pl.ds(base, NCH)], nbuf)
    pltpu.sync_copy(nids.at[pl.ds(ncid * 10, 10)], nidb)

    def ngate(j, carry):
      acc = jnp.zeros((16,), jnp.float32)
      rows = []
      for cc in range(8):
        rr = nbuf[j, cc * 16:(cc + 1) * 16]
        rows.append(rr)
        acc = acc + rr * wgnv[cc]
      a = jnp.sum(acc)
      for cc in range(8):
        ngbuf[j, cc * 16:(cc + 1) * 16] = rows[cc] * a + rows[cc] * bgnv
      return carry

    lax.fori_loop(0, NCH, ngate, 0)
    for r in range(10):
      pltpu.sync_copy(ngbuf.at[pl.ds(r * 16, 16)],
                      npool.at[nidb.at[r]], add=True)

  plsc.subcore_barrier()

  @pl.when(s == 0)
  def _writeout():
    pltpu.sync_copy(npool, npart.at[c])
    pltpu.sync_copy(epool, epart.at[c])


_sc_pool = functools.partial(
    pl.kernel,
    out_type=(jax.ShapeDtypeStruct((NC, G, ND), jnp.float32),
              jax.ShapeDtypeStruct((NC, G, ED), jnp.float32)),
    mesh=plsc.VectorSubcoreMesh(core_axis_name="c", subcore_axis_name="s"),
    scratch_types=(
        pltpu.VMEM((EC, ED), jnp.float32),
        pltpu.VMEM((EC, ED), jnp.float32),
        pltpu.VMEM((10, 128), jnp.int32),
        pltpu.VMEM((NCH, ND), jnp.float32),
        pltpu.VMEM((NCH, ND), jnp.float32),
        pltpu.VMEM((10, 16), jnp.int32),
        pltpu.VMEM((16,), jnp.float32),
        pltpu.VMEM((16,), jnp.float32),
        pltpu.VMEM((128,), jnp.float32),
        pltpu.VMEM((16,), jnp.float32),
        pltpu.VMEM_SHARED((G, ND), jnp.float32),
        pltpu.VMEM_SHARED((G, ED), jnp.float32),
    ),
)(_sc_body)


def _tc_finish_body(np_ref, ep_ref, wpn_ref, wpe_ref, bp_ref, o_ref):
  pooled_n = np_ref[0] + np_ref[1]
  pooled_e = ep_ref[0] + ep_ref[1]
  o_ref[...] = (
      jnp.dot(pooled_n, wpn_ref[...], preferred_element_type=jnp.float32)
      + jnp.dot(pooled_e, wpe_ref[...], preferred_element_type=jnp.float32)
      + bp_ref[...])


_tc_finish = pl.pallas_call(
    _tc_finish_body,
    out_shape=jax.ShapeDtypeStruct((G, PD), jnp.float32),
)


def kernel(node_features, edge_features, node_batch_list, edge_batch_list,
           Wg_n, bg_n, Wg_e, bg_e, Wp, bp):
  pad = N_PAD - node_features.shape[0]
  nf = jnp.concatenate(
      [node_features, jnp.zeros((pad, ND), jnp.float32)], axis=0)
  nids = jnp.concatenate(
      [node_batch_list.astype(jnp.int32),
       jnp.full((pad,), G - 1, jnp.int32)]).reshape(N_PAD // 16, 16)
  eids = edge_batch_list.astype(jnp.int32).reshape(E // 128, 128)
  wge = Wg_e.reshape(ED)
  wgn = Wg_n.reshape(ND)
  bge = jnp.full((16,), bg_e[0], jnp.float32)
  bgn = jnp.full((16,), bg_n[0], jnp.float32)

  npart, epart = _sc_pool(edge_features, eids, nf, nids, wge, bge, wgn, bgn)
  return _tc_finish(npart, epart, Wp[:ND], Wp[ND:], bp.reshape(1, PD))


# speedup vs baseline: 3.0229x; 3.0229x over previous
import functools

import jax
import jax.numpy as jnp
from jax import lax
from jax.experimental import pallas as pl
from jax.experimental.pallas import tpu as pltpu
from jax.experimental.pallas import tpu_sc as plsc

G = 256
ND = 128
ED = 16
PD = 128
N = 10000
N_PAD = 10240
E = 320000
E_PAD = 327680

NC, NS = 2, 16
NW = NC * NS

EC = 1024
E_CHUNKS = E_PAD // EC
E_K = E_CHUNKS // NW
NCH = 128
N_CHUNKS = N_PAD // NCH
N_K = 3


def _lanesum16(x):
  idx = lax.iota(jnp.int32, 16)
  for sh in (8, 4, 2, 1):
    x = x + x.at[jnp.bitwise_xor(idx, sh)].get(mode="promise_in_bounds")
  return x


def _sc_body(ef, eids, nf, nids, wge, bge, wgn, bgn,
             npart, epart,
             ebuf, egbuf, eidb, nbuf, ngbuf, nidb,
             wgeb, bgeb, wgnb, bgnb,
             npool, epool):
  c = lax.axis_index("c")
  s = lax.axis_index("s")
  wid = s * NC + c

  pltpu.sync_copy(wge, wgeb)
  pltpu.sync_copy(bge, bgeb)
  pltpu.sync_copy(wgn, wgnb)
  pltpu.sync_copy(bgn, bgnb)

  @pl.when(s == 0)
  def _zero():
    zero16 = jnp.zeros((16,), jnp.float32)

    def zrow_n(i, carry):
      for cc in range(8):
        ngbuf[i, cc * 16:(cc + 1) * 16] = zero16
      return carry

    lax.fori_loop(0, NCH, zrow_n, 0)


    def zrow_e(i, carry):
      egbuf[i, :] = zero16
      return carry

    lax.fori_loop(0, G, zrow_e, 0)

    pltpu.sync_copy(ngbuf, npool.at[pl.ds(0, NCH)])
    pltpu.sync_copy(ngbuf, npool.at[pl.ds(NCH, NCH)])
    pltpu.sync_copy(egbuf.at[pl.ds(0, G)], epool)

  plsc.subcore_barrier()

  wgev = wgeb[:]
  bgev = bgeb[:]

  for k in range(E_K):
    cid = wid * E_K + k
    base = cid * EC
    pltpu.sync_copy(ef.at[pl.ds(base, EC)], ebuf)
    pltpu.sync_copy(eids.at[pl.ds(cid * 8, 8)], eidb)

    def gate(j, carry):
      row = ebuf[j, :]
      a = _lanesum16(row * wgev) + bgev
      egbuf[j, :] = row * a
      return carry

    lax.fori_loop(0, EC, gate, 0)
    for r in range(8):
      pltpu.sync_copy(egbuf.at[pl.ds(r * 128, 128)],
                      epool.at[eidb.at[r]], add=True)

  wgnv = [wgnb[cc * 16:(cc + 1) * 16] for cc in range(8)]
  bgnv = bgnb[:]
  for k in range(N_K):
    ncid = wid + NW * k

    @pl.when(ncid < N_CHUNKS)
    def _node_chunk():
      base = ncid * NCH
      pltpu.sync_copy(nf.at[pl.ds(base, NCH)], nbuf)
      pltpu.sync_copy(nids.at[pl.ds(ncid * 8, 8)], nidb)

      def ngate(j, carry):
        acc = jnp.zeros((16,), jnp.float32)
        rows = []
        for cc in range(8):
          rr = nbuf[j, cc * 16:(cc + 1) * 16]
          rows.append(rr)
          acc = acc + rr * wgnv[cc]
        a = _lanesum16(acc) + bgnv
        for cc in range(8):
          ngbuf[j, cc * 16:(cc + 1) * 16] = rows[cc] * a
        return carry

      lax.fori_loop(0, NCH, ngate, 0)
      for r in range(8):
        pltpu.sync_copy(ngbuf.at[pl.ds(r * 16, 16)],
                        npool.at[nidb.at[r]], add=True)

  plsc.subcore_barrier()

  @pl.when(s == 0)
  def _writeout():
    pltpu.sync_copy(npool, npart.at[c])
    pltpu.sync_copy(epool, epart.at[c])


_sc_pool = functools.partial(
    pl.kernel,
    out_type=(jax.ShapeDtypeStruct((NC, G, ND), jnp.float32),
              jax.ShapeDtypeStruct((NC, G, ED), jnp.float32)),
    mesh=plsc.VectorSubcoreMesh(core_axis_name="c", subcore_axis_name="s"),
    compiler_params=pltpu.CompilerParams(use_tc_tiling_on_sc=False),
    scratch_types=(
        pltpu.VMEM((EC, ED), jnp.float32),
        pltpu.VMEM((EC, ED), jnp.float32),
        pltpu.VMEM((8, 128), jnp.int32),
        pltpu.VMEM((NCH, ND), jnp.float32),
        pltpu.VMEM((NCH, ND), jnp.float32),
        pltpu.VMEM((8, 16), jnp.int32),
        pltpu.VMEM((16,), jnp.float32),
        pltpu.VMEM((16,), jnp.float32),
        pltpu.VMEM((128,), jnp.float32),
        pltpu.VMEM((16,), jnp.float32),
        pltpu.VMEM_SHARED((G, ND), jnp.float32),
        pltpu.VMEM_SHARED((G, ED), jnp.float32),
    ),
)(_sc_body)


def _tc_finish_body(np_ref, ep_ref, wpn_ref, wpe_ref, bp_ref, o_ref):
  pooled_n = np_ref[0] + np_ref[1]
  pooled_e = ep_ref[0] + ep_ref[1]
  o_ref[...] = (
      jnp.dot(pooled_n, wpn_ref[...], preferred_element_type=jnp.float32)
      + jnp.dot(pooled_e, wpe_ref[...], preferred_element_type=jnp.float32)
      + bp_ref[...])


_tc_finish = pl.pallas_call(
    _tc_finish_body,
    out_shape=jax.ShapeDtypeStruct((G, PD), jnp.float32),
)


def kernel(node_features, edge_features, node_batch_list, edge_batch_list,
           Wg_n, bg_n, Wg_e, bg_e, Wp, bp):
  pad = N_PAD - node_features.shape[0]
  nf = jnp.concatenate(
      [node_features, jnp.zeros((pad, ND), jnp.float32)], axis=0)
  nids = jnp.concatenate(
      [node_batch_list.astype(jnp.int32),
       jnp.full((pad,), G - 1, jnp.int32)]).reshape(N_PAD // 16, 16)
  epad = E_PAD - edge_features.shape[0]
  ef = jnp.concatenate(
      [edge_features, jnp.zeros((epad, ED), jnp.float32)], axis=0)
  eids = jnp.concatenate(
      [edge_batch_list.astype(jnp.int32),
       jnp.full((epad,), G - 1, jnp.int32)]).reshape(E_PAD // 128, 128)
  wge = Wg_e.reshape(ED)
  wgn = Wg_n.reshape(ND)
  bge = jnp.full((16,), bg_e[0], jnp.float32)
  bgn = jnp.full((16,), bg_n[0], jnp.float32)

  npart, epart = _sc_pool(ef, eids, nf, nids, wge, bge, wgn, bgn)
  return _tc_finish(npart, epart, Wp[:ND], Wp[ND:], bp.reshape(1, PD))


# speedup vs baseline: 3.1268x; 1.0344x over previous
import functools

import jax
import jax.numpy as jnp
from jax import lax
from jax.experimental import pallas as pl
from jax.experimental.pallas import tpu as pltpu
from jax.experimental.pallas import tpu_sc as plsc

G = 256
ND = 128
ED = 16
PD = 128
N = 10000
N_PAD = 10240
E = 320000
E_PAD = 327680

NC, NS = 2, 16
NW = NC * NS

EC = 1024
E_CHUNKS = E_PAD // EC
E_K = E_CHUNKS // NW
NCH = 128
N_CHUNKS = N_PAD // NCH
N_K = 3


def _lanesum16(x):
  idx = lax.iota(jnp.int32, 16)
  for sh in (8, 4, 2, 1):
    x = x + x.at[jnp.bitwise_xor(idx, sh)].get(mode="promise_in_bounds")
  return x


def _sc_body(ef, eids, nf, nids, wge, bge, wgn, bgn,
             npart, epart,
             ebuf, egbuf, eidb, nbuf, ngbuf, nidb,
             wgeb, bgeb, wgnb, bgnb,
             npool, epool):
  c = lax.axis_index("c")
  s = lax.axis_index("s")
  wid = s * NC + c

  pltpu.sync_copy(wge, wgeb)
  pltpu.sync_copy(bge, bgeb)
  pltpu.sync_copy(wgn, wgnb)
  pltpu.sync_copy(bgn, bgnb)

  @pl.when(s == 0)
  def _zero():
    zero16 = jnp.zeros((16,), jnp.float32)

    def zrow_n(i, carry):
      for cc in range(8):
        ngbuf[i, cc * 16:(cc + 1) * 16] = zero16
      return carry

    lax.fori_loop(0, NCH, zrow_n, 0)


    def zrow_e(i, carry):
      egbuf[i, :] = zero16
      return carry

    lax.fori_loop(0, G, zrow_e, 0)

    pltpu.sync_copy(ngbuf, npool.at[pl.ds(0, NCH)])
    pltpu.sync_copy(ngbuf, npool.at[pl.ds(NCH, NCH)])
    pltpu.sync_copy(egbuf.at[pl.ds(0, G)], epool)

  plsc.subcore_barrier()

  wgev = wgeb[:]
  bgev = bgeb[:]

  for k in range(E_K):
    cid = wid * E_K + k
    base = cid * EC
    pltpu.sync_copy(ef.at[pl.ds(base, EC)], ebuf)
    pltpu.sync_copy(eids.at[pl.ds(cid * 8, 8)], eidb)

    def gate(g, carry):
      j = g * 8
      rows = [ebuf[j + u, :] for u in range(8)]
      alphas = [_lanesum16(rows[u] * wgev) + bgev for u in range(8)]
      for u in range(8):
        egbuf[j + u, :] = rows[u] * alphas[u]
      return carry

    lax.fori_loop(0, EC // 8, gate, 0)
    for r in range(8):
      pltpu.sync_copy(egbuf.at[pl.ds(r * 128, 128)],
                      epool.at[eidb.at[r]], add=True)

  wgnv = [wgnb[cc * 16:(cc + 1) * 16] for cc in range(8)]
  bgnv = bgnb[:]
  for k in range(N_K):
    ncid = wid + NW * k

    @pl.when(ncid < N_CHUNKS)
    def _node_chunk():
      base = ncid * NCH
      pltpu.sync_copy(nf.at[pl.ds(base, NCH)], nbuf)
      pltpu.sync_copy(nids.at[pl.ds(ncid * 8, 8)], nidb)

      def ngate(g, carry):
        for u in range(2):
          j = g * 2 + u
          acc = jnp.zeros((16,), jnp.float32)
          rows = []
          for cc in range(8):
            rr = nbuf[j, cc * 16:(cc + 1) * 16]
            rows.append(rr)
            acc = acc + rr * wgnv[cc]
          a = _lanesum16(acc) + bgnv
          for cc in range(8):
            ngbuf[j, cc * 16:(cc + 1) * 16] = rows[cc] * a
        return carry

      lax.fori_loop(0, NCH // 2, ngate, 0)
      for r in range(8):
        pltpu.sync_copy(ngbuf.at[pl.ds(r * 16, 16)],
                        npool.at[nidb.at[r]], add=True)

  plsc.subcore_barrier()

  @pl.when(s == 0)
  def _writeout():
    pltpu.sync_copy(npool, npart.at[c])
    pltpu.sync_copy(epool, epart.at[c])


_sc_pool = functools.partial(
    pl.kernel,
    out_type=(jax.ShapeDtypeStruct((NC, G, ND), jnp.float32),
              jax.ShapeDtypeStruct((NC, G, ED), jnp.float32)),
    mesh=plsc.VectorSubcoreMesh(core_axis_name="c", subcore_axis_name="s"),
    compiler_params=pltpu.CompilerParams(use_tc_tiling_on_sc=False),
    scratch_types=(
        pltpu.VMEM((EC, ED), jnp.float32),
        pltpu.VMEM((EC, ED), jnp.float32),
        pltpu.VMEM((8, 128), jnp.int32),
        pltpu.VMEM((NCH, ND), jnp.float32),
        pltpu.VMEM((NCH, ND), jnp.float32),
        pltpu.VMEM((8, 16), jnp.int32),
        pltpu.VMEM((16,), jnp.float32),
        pltpu.VMEM((16,), jnp.float32),
        pltpu.VMEM((128,), jnp.float32),
        pltpu.VMEM((16,), jnp.float32),
        pltpu.VMEM_SHARED((G, ND), jnp.float32),
        pltpu.VMEM_SHARED((G, ED), jnp.float32),
    ),
)(_sc_body)


def _tc_finish_body(np_ref, ep_ref, wpn_ref, wpe_ref, bp_ref, o_ref):
  pooled_n = np_ref[0] + np_ref[1]
  pooled_e = ep_ref[0] + ep_ref[1]
  o_ref[...] = (
      jnp.dot(pooled_n, wpn_ref[...], preferred_element_type=jnp.float32)
      + jnp.dot(pooled_e, wpe_ref[...], preferred_element_type=jnp.float32)
      + bp_ref[...])


_tc_finish = pl.pallas_call(
    _tc_finish_body,
    out_shape=jax.ShapeDtypeStruct((G, PD), jnp.float32),
)


def kernel(node_features, edge_features, node_batch_list, edge_batch_list,
           Wg_n, bg_n, Wg_e, bg_e, Wp, bp):
  pad = N_PAD - node_features.shape[0]
  nf = jnp.concatenate(
      [node_features, jnp.zeros((pad, ND), jnp.float32)], axis=0)
  nids = jnp.concatenate(
      [node_batch_list.astype(jnp.int32),
       jnp.full((pad,), G - 1, jnp.int32)]).reshape(N_PAD // 16, 16)
  epad = E_PAD - edge_features.shape[0]
  ef = jnp.concatenate(
      [edge_features, jnp.zeros((epad, ED), jnp.float32)], axis=0)
  eids = jnp.concatenate(
      [edge_batch_list.astype(jnp.int32),
       jnp.full((epad,), G - 1, jnp.int32)]).reshape(E_PAD // 128, 128)
  wge = Wg_e.reshape(ED)
  wgn = Wg_n.reshape(ND)
  bge = jnp.full((16,), bg_e[0], jnp.float32)
  bgn = jnp.full((16,), bg_n[0], jnp.float32)

  npart, epart = _sc_pool(ef, eids, nf, nids, wge, bge, wgn, bgn)
  return _tc_finish(npart, epart, Wp[:ND], Wp[ND:], bp.reshape(1, PD))


# speedup vs baseline: 4.6887x; 1.4995x over previous
import functools

import jax
import jax.numpy as jnp
from jax import lax
from jax.experimental import pallas as pl
from jax.experimental.pallas import tpu as pltpu
from jax.experimental.pallas import tpu_sc as plsc

G = 256
ND = 128
ED = 16
PD = 128
N = 10000
E = 320000

NC, NS = 2, 16
NW = NC * NS

EC = 1024
E_FULL = E // EC
E_TAIL = E - E_FULL * EC
E_K = (E_FULL + NW - 1) // NW
NCH = 128
N_FULL = N // NCH
N_TAIL = N - N_FULL * NCH
N_K = (N_FULL + NW - 1) // NW


def _lanesum16(x):
  idx = lax.iota(jnp.int32, 16)
  for sh in (8, 4, 2, 1):
    x = x + x.at[jnp.bitwise_xor(idx, sh)].get(mode="promise_in_bounds")
  return x


def _sc_body(ef, eids, nf, nids, wge, bge, wgn, bgn,
             npart, epart,
             ebuf, egbuf, eidb, nbuf, ngbuf, nidb,
             wgeb, bgeb, wgnb, bgnb,
             npool, epool):
  c = lax.axis_index("c")
  s = lax.axis_index("s")
  wid = s * NC + c

  pltpu.sync_copy(wge, wgeb)
  pltpu.sync_copy(bge, bgeb)
  pltpu.sync_copy(wgn, wgnb)
  pltpu.sync_copy(bgn, bgnb)

  @pl.when(s == 0)
  def _zero():
    zero16 = jnp.zeros((16,), jnp.float32)

    def zrow_n(i, carry):
      for cc in range(8):
        ngbuf[i, cc * 16:(cc + 1) * 16] = zero16
      return carry

    lax.fori_loop(0, NCH, zrow_n, 0)


    def zrow_e(i, carry):
      egbuf[i, :] = zero16
      return carry

    lax.fori_loop(0, G, zrow_e, 0)

    pltpu.sync_copy(ngbuf, npool.at[pl.ds(0, NCH)])
    pltpu.sync_copy(ngbuf, npool.at[pl.ds(NCH, NCH)])
    pltpu.sync_copy(egbuf.at[pl.ds(0, G)], epool)

  plsc.subcore_barrier()

  wgev = wgeb[:]
  bgev = bgeb[:]

  def edge_chunk(base, n_edges):
    pltpu.sync_copy(ef.at[pl.ds(base, n_edges)], ebuf.at[pl.ds(0, n_edges)])
    pltpu.sync_copy(eids.at[pl.ds(base, n_edges)], eidb.at[pl.ds(0, n_edges)])

    def gate(g, carry):
      j = g * 8
      rows = [ebuf[j + u, :] for u in range(8)]
      alphas = [_lanesum16(rows[u] * wgev) + bgev for u in range(8)]
      for u in range(8):
        egbuf[j + u, :] = rows[u] * alphas[u]
      return carry

    lax.fori_loop(0, n_edges // 8, gate, 0)
    for r in range(n_edges // 128):
      pltpu.sync_copy(egbuf.at[pl.ds(r * 128, 128)],
                      epool.at[eidb.at[pl.ds(r * 128, 128)]], add=True)

  for k in range(E_K):
    cid = wid + NW * k

    @pl.when(cid < E_FULL)
    def _full():
      edge_chunk(cid * EC, EC)

  @pl.when(wid == NW - 1)
  def _etail():
    edge_chunk(E_FULL * EC, E_TAIL)

  wgnv = [wgnb[cc * 16:(cc + 1) * 16] for cc in range(8)]
  bgnv = bgnb[:]
  def node_chunk(base, n_nodes):
    pltpu.sync_copy(nf.at[pl.ds(base, n_nodes)], nbuf.at[pl.ds(0, n_nodes)])
    pltpu.sync_copy(nids.at[pl.ds(base, n_nodes)], nidb.at[pl.ds(0, n_nodes)])

    def ngate(g, carry):
      for u in range(2):
        j = g * 2 + u
        acc = jnp.zeros((16,), jnp.float32)
        rows = []
        for cc in range(8):
          rr = nbuf[j, cc * 16:(cc + 1) * 16]
          rows.append(rr)
          acc = acc + rr * wgnv[cc]
        a = _lanesum16(acc) + bgnv
        for cc in range(8):
          ngbuf[j, cc * 16:(cc + 1) * 16] = rows[cc] * a
      return carry

    lax.fori_loop(0, n_nodes // 2, ngate, 0)
    for r in range(n_nodes // 16):
      pltpu.sync_copy(ngbuf.at[pl.ds(r * 16, 16)],
                      npool.at[nidb.at[pl.ds(r * 16, 16)]], add=True)

  for k in range(N_K):
    ncid = wid + NW * k

    @pl.when(ncid < N_FULL)
    def _node_full():
      node_chunk(ncid * NCH, NCH)

  @pl.when(wid == NW - 2)
  def _ntail():
    node_chunk(N_FULL * NCH, N_TAIL)

  plsc.subcore_barrier()

  @pl.when(s == 0)
  def _writeout():
    pltpu.sync_copy(npool, npart.at[c])
    pltpu.sync_copy(epool, epart.at[c])


_sc_pool = functools.partial(
    pl.kernel,
    out_type=(jax.ShapeDtypeStruct((NC, G, ND), jnp.float32),
              jax.ShapeDtypeStruct((NC, G, ED), jnp.float32)),
    mesh=plsc.VectorSubcoreMesh(core_axis_name="c", subcore_axis_name="s"),
    compiler_params=pltpu.CompilerParams(use_tc_tiling_on_sc=False),
    scratch_types=(
        pltpu.VMEM((EC, ED), jnp.float32),
        pltpu.VMEM((EC, ED), jnp.float32),
        pltpu.VMEM((EC,), jnp.int32),
        pltpu.VMEM((NCH, ND), jnp.float32),
        pltpu.VMEM((NCH, ND), jnp.float32),
        pltpu.VMEM((NCH,), jnp.int32),
        pltpu.VMEM((16,), jnp.float32),
        pltpu.VMEM((16,), jnp.float32),
        pltpu.VMEM((128,), jnp.float32),
        pltpu.VMEM((16,), jnp.float32),
        pltpu.VMEM_SHARED((G, ND), jnp.float32),
        pltpu.VMEM_SHARED((G, ED), jnp.float32),
    ),
)(_sc_body)


def _tc_finish_body(np_ref, ep_ref, wpn_ref, wpe_ref, bp_ref, o_ref):
  pooled_n = np_ref[0] + np_ref[1]
  pooled_e = ep_ref[0] + ep_ref[1]
  o_ref[...] = (
      jnp.dot(pooled_n, wpn_ref[...], preferred_element_type=jnp.float32)
      + jnp.dot(pooled_e, wpe_ref[...], preferred_element_type=jnp.float32)
      + bp_ref[...])


_tc_finish = pl.pallas_call(
    _tc_finish_body,
    out_shape=jax.ShapeDtypeStruct((G, PD), jnp.float32),
)


def kernel(node_features, edge_features, node_batch_list, edge_batch_list,
           Wg_n, bg_n, Wg_e, bg_e, Wp, bp):
  nids = node_batch_list.astype(jnp.int32)
  eids = edge_batch_list.astype(jnp.int32)
  wge = Wg_e.reshape(ED)
  wgn = Wg_n.reshape(ND)
  bge = jnp.full((16,), bg_e[0], jnp.float32)
  bgn = jnp.full((16,), bg_n[0], jnp.float32)

  npart, epart = _sc_pool(edge_features, eids, node_features, nids,
                          wge, bge, wgn, bgn)
  return _tc_finish(npart, epart, Wp[:ND], Wp[ND:], bp.reshape(1, PD))


# speedup vs baseline: 4.6979x; 1.0020x over previous
import functools

import jax
import jax.numpy as jnp
from jax import lax
from jax.experimental import pallas as pl
from jax.experimental.pallas import tpu as pltpu
from jax.experimental.pallas import tpu_sc as plsc

G = 256
ND = 128
ED = 16
PD = 128
N = 10000
E = 320000

NC, NS = 2, 16
NW = NC * NS

EC = 1024
E_FULL = E // EC
E_TAIL = E - E_FULL * EC
E_K = (E_FULL + NW - 1) // NW
NCH = 128
N_FULL = N // NCH
N_TAIL = N - N_FULL * NCH
N_K = (N_FULL + NW - 1) // NW


def _lanesum16(x):
  idx = lax.iota(jnp.int32, 16)
  for sh in (8, 4, 2, 1):
    x = x + x.at[jnp.bitwise_xor(idx, sh)].get(mode="promise_in_bounds")
  return x


def _sc_body(ef, eids, nf, nids, wge, bge, wgn, bgn,
             npart, epart,
             ebuf, egbuf, eidb, nbuf, ngbuf, nidb,
             wgeb, bgeb, wgnb, bgnb,
             npool, epool):
  c = lax.axis_index("c")
  s = lax.axis_index("s")
  wid = s * NC + c

  pltpu.sync_copy(wge, wgeb)
  pltpu.sync_copy(bge, bgeb)
  pltpu.sync_copy(wgn, wgnb)
  pltpu.sync_copy(bgn, bgnb)

  @pl.when(s == 0)
  def _zero():
    zero16 = jnp.zeros((16,), jnp.float32)

    def zrow_n(i, carry):
      for cc in range(8):
        ngbuf[i, cc * 16:(cc + 1) * 16] = zero16
      return carry

    lax.fori_loop(0, NCH, zrow_n, 0)


    def zrow_e(i, carry):
      egbuf[i, :] = zero16
      return carry

    lax.fori_loop(0, G, zrow_e, 0)

    pltpu.sync_copy(ngbuf, npool.at[pl.ds(0, NCH)])
    pltpu.sync_copy(ngbuf, npool.at[pl.ds(NCH, NCH)])
    pltpu.sync_copy(egbuf.at[pl.ds(0, G)], epool)

  plsc.subcore_barrier()

  wgev = wgeb[:]
  bgev = bgeb[:]

  def edge_chunk(base, n_edges):
    nr = n_edges // 8
    pltpu.sync_copy(ef.at[pl.ds(base // 8, nr)], ebuf.at[pl.ds(0, nr)])
    pltpu.sync_copy(eids.at[pl.ds(base, n_edges)], eidb.at[pl.ds(0, n_edges)])

    def gate(g, carry):
      j = g * 8
      rows = [ebuf[g, u * 16:(u + 1) * 16] for u in range(8)]
      alphas = [_lanesum16(rows[u] * wgev) + bgev for u in range(8)]
      for u in range(8):
        egbuf[j + u, :] = rows[u] * alphas[u]
      return carry

    lax.fori_loop(0, nr, gate, 0)
    for r in range(n_edges // 128):
      pltpu.sync_copy(egbuf.at[pl.ds(r * 128, 128)],
                      epool.at[eidb.at[pl.ds(r * 128, 128)]], add=True)

  for k in range(E_K):
    cid = wid + NW * k

    @pl.when(cid < E_FULL)
    def _full():
      edge_chunk(cid * EC, EC)

  @pl.when(wid == NW - 1)
  def _etail():
    edge_chunk(E_FULL * EC, E_TAIL)

  wgnv = [wgnb[cc * 16:(cc + 1) * 16] for cc in range(8)]
  bgnv = bgnb[:]
  def node_chunk(base, n_nodes):
    pltpu.sync_copy(nf.at[pl.ds(base, n_nodes)], nbuf.at[pl.ds(0, n_nodes)])
    pltpu.sync_copy(nids.at[pl.ds(base, n_nodes)], nidb.at[pl.ds(0, n_nodes)])

    def ngate(g, carry):
      for u in range(2):
        j = g * 2 + u
        acc = jnp.zeros((16,), jnp.float32)
        rows = []
        for cc in range(8):
          rr = nbuf[j, cc * 16:(cc + 1) * 16]
          rows.append(rr)
          acc = acc + rr * wgnv[cc]
        a = _lanesum16(acc) + bgnv
        for cc in range(8):
          ngbuf[j, cc * 16:(cc + 1) * 16] = rows[cc] * a
      return carry

    lax.fori_loop(0, n_nodes // 2, ngate, 0)
    for r in range(n_nodes // 16):
      pltpu.sync_copy(ngbuf.at[pl.ds(r * 16, 16)],
                      npool.at[nidb.at[pl.ds(r * 16, 16)]], add=True)

  for k in range(N_K):
    ncid = wid + NW * k

    @pl.when(ncid < N_FULL)
    def _node_full():
      node_chunk(ncid * NCH, NCH)

  @pl.when(wid == NW - 2)
  def _ntail():
    node_chunk(N_FULL * NCH, N_TAIL)

  plsc.subcore_barrier()

  @pl.when(s == 0)
  def _writeout():
    pltpu.sync_copy(npool, npart.at[c])
    pltpu.sync_copy(epool, epart.at[c])


_sc_pool = functools.partial(
    pl.kernel,
    out_type=(jax.ShapeDtypeStruct((NC, G, ND), jnp.float32),
              jax.ShapeDtypeStruct((NC, G, ED), jnp.float32)),
    mesh=plsc.VectorSubcoreMesh(core_axis_name="c", subcore_axis_name="s"),
    compiler_params=pltpu.CompilerParams(use_tc_tiling_on_sc=False),
    scratch_types=(
        pltpu.VMEM((EC // 8, 128), jnp.float32),
        pltpu.VMEM((EC, ED), jnp.float32),
        pltpu.VMEM((EC,), jnp.int32),
        pltpu.VMEM((NCH, ND), jnp.float32),
        pltpu.VMEM((NCH, ND), jnp.float32),
        pltpu.VMEM((NCH,), jnp.int32),
        pltpu.VMEM((16,), jnp.float32),
        pltpu.VMEM((16,), jnp.float32),
        pltpu.VMEM((128,), jnp.float32),
        pltpu.VMEM((16,), jnp.float32),
        pltpu.VMEM_SHARED((G, ND), jnp.float32),
        pltpu.VMEM_SHARED((G, ED), jnp.float32),
    ),
)(_sc_body)


def _tc_finish_body(np_ref, ep_ref, wpn_ref, wpe_ref, bp_ref, o_ref):
  pooled_n = np_ref[0] + np_ref[1]
  pooled_e = ep_ref[0] + ep_ref[1]
  o_ref[...] = (
      jnp.dot(pooled_n, wpn_ref[...], preferred_element_type=jnp.float32)
      + jnp.dot(pooled_e, wpe_ref[...], preferred_element_type=jnp.float32)
      + bp_ref[...])


_tc_finish = pl.pallas_call(
    _tc_finish_body,
    out_shape=jax.ShapeDtypeStruct((G, PD), jnp.float32),
)


def kernel(node_features, edge_features, node_batch_list, edge_batch_list,
           Wg_n, bg_n, Wg_e, bg_e, Wp, bp):
  nids = node_batch_list.astype(jnp.int32)
  eids = edge_batch_list.astype(jnp.int32)
  wge = Wg_e.reshape(ED)
  wgn = Wg_n.reshape(ND)
  bge = jnp.full((16,), bg_e[0], jnp.float32)
  bgn = jnp.full((16,), bg_n[0], jnp.float32)

  ef8 = edge_features.reshape(E // 8, 8 * ED)
  npart, epart = _sc_pool(ef8, eids, node_features, nids,
                          wge, bge, wgn, bgn)
  return _tc_finish(npart, epart, Wp[:ND], Wp[ND:], bp.reshape(1, PD))
